# Initial kernel scaffold; baseline (speedup 1.0000x reference)
import jax
import jax.numpy as jnp
from jax.experimental import pallas as pl


def kernel(x, pos_embed):
    raise NotImplementedError("write your pallas kernel here")


# speedup vs baseline: 1.6806x; 1.6806x over previous
import jax
import jax.numpy as jnp
from jax.experimental import pallas as pl


BN = 512


def _add_kernel(x_ref, pos_ref, o_ref):
    o_ref[...] = x_ref[...] + pos_ref[...]


def kernel(x, pos_embed):
    B, N, D = x.shape
    grid = (N // BN, B)
    return pl.pallas_call(
        _add_kernel,
        grid=grid,
        in_specs=[
            pl.BlockSpec((1, BN, D), lambda j, b: (b, j, 0)),
            pl.BlockSpec((BN, D), lambda j, b: (j, 0)),
        ],
        out_specs=pl.BlockSpec((1, BN, D), lambda j, b: (b, j, 0)),
        out_shape=jax.ShapeDtypeStruct((B, N, D), x.dtype),
    )(x, pos_embed)


# speedup vs baseline: 1.8716x; 1.1137x over previous
import jax
import jax.numpy as jnp
from jax.experimental import pallas as pl


BN = 1024


def _add_kernel(x_ref, pos_ref, o_ref):
    o_ref[...] = x_ref[...] + pos_ref[...]


def kernel(x, pos_embed):
    B, N, D = x.shape
    grid = (N // BN, B)
    return pl.pallas_call(
        _add_kernel,
        grid=grid,
        in_specs=[
            pl.BlockSpec((1, BN, D), lambda j, b: (b, j, 0)),
            pl.BlockSpec((BN, D), lambda j, b: (j, 0)),
        ],
        out_specs=pl.BlockSpec((1, BN, D), lambda j, b: (b, j, 0)),
        out_shape=jax.ShapeDtypeStruct((B, N, D), x.dtype),
    )(x, pos_embed)


# speedup vs baseline: 1.9925x; 1.0646x over previous
import jax
import jax.numpy as jnp
from jax.experimental import pallas as pl


BN = 2048


def _add_kernel(x_ref, pos_ref, o_ref):
    o_ref[...] = x_ref[...] + pos_ref[...]


def kernel(x, pos_embed):
    B, N, D = x.shape
    grid = (N // BN, B)
    return pl.pallas_call(
        _add_kernel,
        grid=grid,
        in_specs=[
            pl.BlockSpec((1, BN, D), lambda j, b: (b, j, 0)),
            pl.BlockSpec((BN, D), lambda j, b: (j, 0)),
        ],
        out_specs=pl.BlockSpec((1, BN, D), lambda j, b: (b, j, 0)),
        out_shape=jax.ShapeDtypeStruct((B, N, D), x.dtype),
    )(x, pos_embed)


# speedup vs baseline: 1.9956x; 1.0016x over previous
import jax
import jax.numpy as jnp
from jax.experimental import pallas as pl
from jax.experimental.pallas import tpu as pltpu


BN = 2048


def _add_kernel(x_ref, pos_ref, o_ref):
    o_ref[...] = x_ref[...] + pos_ref[...]


def kernel(x, pos_embed):
    B, N, D = x.shape
    grid = (N // BN, B)
    return pl.pallas_call(
        _add_kernel,
        grid=grid,
        in_specs=[
            pl.BlockSpec((1, BN, D), lambda j, b: (b, j, 0)),
            pl.BlockSpec((BN, D), lambda j, b: (j, 0)),
        ],
        out_specs=pl.BlockSpec((1, BN, D), lambda j, b: (b, j, 0)),
        out_shape=jax.ShapeDtypeStruct((B, N, D), x.dtype),
        compiler_params=pltpu.CompilerParams(
            dimension_semantics=("parallel", "parallel")
        ),
    )(x, pos_embed)
